# Initial kernel scaffold; baseline (speedup 1.0000x reference)
import jax
import jax.numpy as jnp
from jax.experimental import pallas as pl


def kernel(mag, integral_m, harmonic_loc, freq_dim):
    raise NotImplementedError("write your pallas kernel here")


# speedup vs baseline: 21.7821x; 21.7821x over previous
import functools

import jax
import jax.numpy as jnp
from jax import lax
from jax.experimental import pallas as pl
from jax.experimental.pallas import tpu as pltpu

_K = 4
_TB = 256


def _fused_body(mag_ref, im_ref, loc_ref, out_ref, carry_ref):
    nt = pl.program_id(1)

    @pl.when(nt == 0)
    def _init():
        carry_ref[...] = jnp.full(carry_ref.shape, 1e-8, jnp.float32)

    magb = mag_ref[0, 0]
    im = im_ref[0, 0]
    loc = loc_ref[0, 0]

    vals = jnp.dot(im.astype(jnp.bfloat16), magb.astype(jnp.bfloat16),
                   preferred_element_type=jnp.float32)
    n, tb = vals.shape
    idx2d = lax.broadcasted_iota(jnp.int32, (n, tb), 0)

    pos_rows = []
    v = vals
    for _ in range(_K):
        mx = jnp.max(v, axis=0)
        cand = jnp.where(v >= mx[None, :], idx2d, jnp.int32(2**30))
        ix = jnp.min(cand, axis=0)
        pos_rows.append(ix)
        v = jnp.where(idx2d == ix[None, :], jnp.float32(-1e30), v)
    posf = jnp.stack(pos_rows, axis=0).astype(jnp.float32)

    carry = carry_ref[0:_K, 0:2]
    pfull = jnp.concatenate([carry, posf], axis=1)
    carry_ref[0:_K, 0:2] = posf[:, tb - 2:tb]
    sm = (pfull[:, 0:tb] + pfull[:, 1:tb + 1] + pfull[:, 2:tb + 2]) / 3.0
    choose = sm.astype(jnp.int32)

    counts = (idx2d == choose[0:1, :]).astype(jnp.float32)
    for i in range(1, _K):
        counts = counts + (idx2d == choose[i:i + 1, :]).astype(jnp.float32)
    g = lax.dot_general(
        loc, counts,
        dimension_numbers=(((0,), (0,)), ((), ())),
        preferred_element_type=jnp.float32,
        precision=lax.Precision.HIGHEST,
    )
    out_ref[0, 0] = (g > 0.0).astype(jnp.float32)


@jax.jit
def _run(mag, integral_m, harmonic_loc):
    B, C, F, T = mag.shape
    N = integral_m.shape[2]
    nt = T // _TB
    return pl.pallas_call(
        _fused_body,
        grid=(B, nt),
        in_specs=[
            pl.BlockSpec((1, 1, F, _TB), lambda b, t: (b, 0, 0, t)),
            pl.BlockSpec((1, 1, N, F), lambda b, t: (0, 0, 0, 0)),
            pl.BlockSpec((1, 1, N, F), lambda b, t: (0, 0, 0, 0)),
        ],
        out_specs=pl.BlockSpec((1, 1, F, _TB), lambda b, t: (b, 0, 0, t)),
        out_shape=jax.ShapeDtypeStruct((B, C, F, T), jnp.float32),
        scratch_shapes=[pltpu.VMEM((8, 128), jnp.float32)],
        compiler_params=pltpu.CompilerParams(
            dimension_semantics=("arbitrary", "arbitrary"),
        ),
    )(mag, integral_m, harmonic_loc)


def kernel(mag, integral_m, harmonic_loc, freq_dim):
    del freq_dim
    return _run(mag, integral_m, harmonic_loc)


# speedup vs baseline: 26.9734x; 1.2383x over previous
import functools

import jax
import jax.numpy as jnp
from jax import lax
from jax.experimental import pallas as pl
from jax.experimental.pallas import tpu as pltpu

_K = 4
_TB = 512


def _fused_body(mag_ref, im_ref, loc_ref, out_ref, carry_ref):
    nt = pl.program_id(1)

    @pl.when(nt == 0)
    def _init():
        carry_ref[...] = jnp.full(carry_ref.shape, 1e-8, jnp.float32)

    magb = mag_ref[0, 0]
    im = im_ref[0, 0]
    loc = loc_ref[0, 0]

    vals = jnp.dot(im.astype(jnp.bfloat16), magb.astype(jnp.bfloat16),
                   preferred_element_type=jnp.float32)
    n, tb = vals.shape
    idx2d = lax.broadcasted_iota(jnp.int32, (n, tb), 0)

    pos_rows = []
    v = vals
    for _ in range(_K):
        mx = jnp.max(v, axis=0)
        cand = jnp.where(v >= mx[None, :], idx2d, jnp.int32(2**30))
        ix = jnp.min(cand, axis=0)
        pos_rows.append(ix)
        v = jnp.where(idx2d == ix[None, :], jnp.float32(-1e30), v)
    posf = jnp.stack(pos_rows, axis=0).astype(jnp.float32)

    carry = carry_ref[0:_K, 0:2]
    pfull = jnp.concatenate([carry, posf], axis=1)
    carry_ref[0:_K, 0:2] = posf[:, tb - 2:tb]
    sm = (pfull[:, 0:tb] + pfull[:, 1:tb + 1] + pfull[:, 2:tb + 2]) / 3.0
    choose = sm.astype(jnp.int32)

    counts = (idx2d == choose[0:1, :]).astype(jnp.float32)
    for i in range(1, _K):
        counts = counts + (idx2d == choose[i:i + 1, :]).astype(jnp.float32)
    g = lax.dot_general(
        loc, counts,
        dimension_numbers=(((0,), (0,)), ((), ())),
        preferred_element_type=jnp.float32,
        precision=lax.Precision.HIGHEST,
    )
    out_ref[0, 0] = (g > 0.0).astype(jnp.float32)


@jax.jit
def _run(mag, integral_m, harmonic_loc):
    B, C, F, T = mag.shape
    N = integral_m.shape[2]
    nt = T // _TB
    return pl.pallas_call(
        _fused_body,
        grid=(B, nt),
        in_specs=[
            pl.BlockSpec((1, 1, F, _TB), lambda b, t: (b, 0, 0, t)),
            pl.BlockSpec((1, 1, N, F), lambda b, t: (0, 0, 0, 0)),
            pl.BlockSpec((1, 1, N, F), lambda b, t: (0, 0, 0, 0)),
        ],
        out_specs=pl.BlockSpec((1, 1, F, _TB), lambda b, t: (b, 0, 0, t)),
        out_shape=jax.ShapeDtypeStruct((B, C, F, T), jnp.float32),
        scratch_shapes=[pltpu.VMEM((8, 128), jnp.float32)],
        compiler_params=pltpu.CompilerParams(
            dimension_semantics=("arbitrary", "arbitrary"),
        ),
    )(mag, integral_m, harmonic_loc)


def kernel(mag, integral_m, harmonic_loc, freq_dim):
    del freq_dim
    return _run(mag, integral_m, harmonic_loc)
